# baseline (speedup 1.0000x reference)
import functools

import jax
import jax.numpy as jnp
from jax import lax
from jax.experimental import pallas as pl
from jax.experimental.pallas import tpu as pltpu
from jax.experimental.pallas import tpu_sc as plsc

N = 10000
E = 320000
D = 128
NC = 2
NS = 16
NW = NC * NS
EPT = E // NW
CH = 128
DH = D // NC
CPB = 160
EPAD = NS * CPB * CH
NB = 4
NI = 8
UN = 8
NPAD = 10240
RPT = NPAD // NS
ZR = 128
DCH = 2000
BN = 1000
GRID = N // BN

_sc_mesh = plsc.VectorSubcoreMesh(core_axis_name="c", subcore_axis_name="s")


@functools.partial(
    pl.kernel,
    out_type=jax.ShapeDtypeStruct((NW, N), jnp.float32),
    mesh=_sc_mesh,
    compiler_params=pltpu.CompilerParams(needs_layout_passes=False),
    scratch_types=[
        pltpu.VMEM((N,), jnp.float32),
        pltpu.VMEM((DCH,), jnp.int32),
        pltpu.VMEM((DCH,), jnp.float32),
    ],
)
def _deg_kernel(col_hbm, w_hbm, out_hbm, acc, colbuf, wbuf):
    cid = lax.axis_index("c")
    sid = lax.axis_index("s")
    wid = cid * NS + sid

    def zero_body(i, _):
        acc[pl.ds(i * 16, 16)] = jnp.zeros((16,), jnp.float32)
        return 0

    lax.fori_loop(0, N // 16, zero_body, 0)

    def chunk_body(i, _):
        base = wid * EPT + i * DCH
        pltpu.sync_copy(col_hbm.at[pl.ds(base, DCH)], colbuf)
        pltpu.sync_copy(w_hbm.at[pl.ds(base, DCH)], wbuf)

        def grp(g, _):
            idx = colbuf[pl.ds(g * 16, 16)]
            val = wbuf[pl.ds(g * 16, 16)]
            plsc.addupdate_scatter(acc, [idx], val)
            return 0

        lax.fori_loop(0, DCH // 16, grp, 0)
        return 0

    lax.fori_loop(0, EPT // DCH, chunk_body, 0)
    pltpu.sync_copy(acc, out_hbm.at[wid])


@functools.partial(
    pl.kernel,
    out_type=jax.ShapeDtypeStruct((NC, NPAD, DH), jnp.float32),
    mesh=_sc_mesh,
    compiler_params=pltpu.CompilerParams(needs_layout_passes=False,
                                         use_tc_tiling_on_sc=False),
    scratch_types=[
        pltpu.VMEM_SHARED((NPAD, DH), jnp.float32),
        pltpu.VMEM((ZR, DH), jnp.float32),
        pltpu.VMEM((NI, 2, CH), jnp.int32),
        pltpu.VMEM((NI, CH), jnp.float32),
        pltpu.VMEM((NB * CH, DH), jnp.float32),
        [pltpu.SemaphoreType.DMA] * NI,
        [pltpu.SemaphoreType.DMA] * NB,
        [pltpu.SemaphoreType.DMA] * NB,
    ],
)
def _agg_kernel(hp0_hbm, hp1_hbm, rc_hbm, w_hbm, out_hbm,
                acc_sh, zbuf, idxring, wring, databuf, isems, gsems, ssems):
    cid = lax.axis_index("c")
    sid = lax.axis_index("s")

    def zb(r, _):
        for f in range(DH // 16):
            zbuf[r, pl.ds(f * 16, 16)] = jnp.zeros((16,), jnp.float32)
        return 0

    lax.fori_loop(0, ZR, zb, 0)
    for k in range(RPT // ZR):
        pltpu.sync_copy(zbuf, acc_sh.at[pl.ds(sid * RPT + k * ZR, ZR)])
    plsc.subcore_barrier()

    def issue_idx(j, m):
        pltpu.async_copy(rc_hbm.at[sid * CPB + j], idxring.at[m], isems[m])
        pltpu.async_copy(w_hbm.at[sid * CPB + j], wring.at[m], isems[m])

    def wait_idx(j, m):
        pltpu.make_async_copy(rc_hbm.at[sid * CPB + j], idxring.at[m],
                              isems[m]).wait()
        pltpu.make_async_copy(w_hbm.at[sid * CPB + j], wring.at[m],
                              isems[m]).wait()

    def dslice(b):
        return databuf.at[pl.ds(b * CH, CH)]

    def issue_gather(m, b):
        @pl.when(cid == 0)
        def _():
            pltpu.async_copy(hp0_hbm.at[idxring.at[m, 0]],
                             dslice(b), gsems[b])

        @pl.when(cid == 1)
        def _():
            pltpu.async_copy(hp1_hbm.at[idxring.at[m, 0]],
                             dslice(b), gsems[b])

    def wait_gather(m, b):
        pltpu.make_async_copy(hp0_hbm.at[idxring.at[m, 0]],
                              dslice(b), gsems[b]).wait()

    def issue_scatter(m, b):
        pltpu.async_copy(dslice(b), acc_sh.at[idxring.at[m, 1]],
                         ssems[b], add=True)

    def wait_scatter(m, b):
        pltpu.make_async_copy(dslice(b), acc_sh.at[idxring.at[m, 1]],
                              ssems[b]).wait()

    def scale_rows(b, m):
        def grp(g, _):
            w16 = wring[m, pl.ds(g * 16, 16)]
            for e in range(16):
                w_s = w16[e]
                r = b * CH + g * 16 + e
                for f in range(DH // 16):
                    v = databuf[r, pl.ds(f * 16, 16)]
                    databuf[r, pl.ds(f * 16, 16)] = v * w_s
            return 0

        lax.fori_loop(0, CH // 16, grp, 0)

    for k in range(4):
        issue_idx(k, k)
    wait_idx(0, 0)
    issue_gather(0, 0)
    wait_idx(1, 1)
    issue_gather(1, 1)

    T = CPB // UN

    def block(t, _):
        for u in range(UN):
            b = u % NB
            f = (u + 2) % NB
            m = u
            mp = (u + 6) % NI
            mg = (u + 2) % NI
            mi = (u + 4) % NI

            def retire_and_gather():
                wait_scatter(mp, f)
                wait_idx(0, mg)
                issue_gather(mg, f)

            if u < 2:
                @pl.when(t >= 1)
                def _():
                    wait_scatter(mp, f)
                wait_idx(0, mg)
                issue_gather(mg, f)
            elif u < 6:
                retire_and_gather()
            else:
                wait_scatter(mp, f)

                @pl.when(t < T - 1)
                def _():
                    wait_idx(0, mg)
                    issue_gather(mg, f)

            j4 = t * UN + u + 4
            if u < 4:
                issue_idx(j4, mi)
            else:
                @pl.when(t < T - 1)
                def _():
                    issue_idx(j4, mi)

            wait_gather(m, b)
            scale_rows(b, m)
            issue_scatter(m, b)
        return 0

    lax.fori_loop(0, T, block, 0)
    wait_scatter((CPB - 2) % NI, (CPB - 2) % NB)
    wait_scatter((CPB - 1) % NI, (CPB - 1) % NB)

    plsc.subcore_barrier()
    pltpu.sync_copy(acc_sh.at[pl.ds(sid * RPT, RPT)],
                    out_hbm.at[cid, pl.ds(sid * RPT, RPT)])


def _dis_from(degp):
    deg = jnp.sum(degp, axis=-1) + 1.0
    return jnp.where(deg > 0, lax.rsqrt(deg), 0.0)


def _tc_prep_body(degp_ref, x_ref, w0_ref, hp0_ref, hp1_ref):
    dis = _dis_from(degp_ref[...])
    h = jnp.dot(x_ref[...], w0_ref[...],
                preferred_element_type=jnp.float32) * dis[:, None]
    hp0_ref[...] = h[:, :DH]
    hp1_ref[...] = h[:, DH:]


def _ln_relu(y, g, be):
    mu = jnp.mean(y, axis=-1, keepdims=True)
    var = jnp.mean((y - mu) ** 2, axis=-1, keepdims=True)
    return jnp.maximum((y - mu) * lax.rsqrt(var + 1e-5) * g + be, 0.0)


def _cat(ref):
    return jnp.concatenate([ref[0], ref[1]], axis=-1)


def _tc_post_body(degp_ref, x_ref, hp0_ref, hp1_ref, acc_ref,
                  b_ref, g_ref, be_ref, wn_ref, wjk_ref, jk_ref,
                  xn_ref, hpn0_ref, hpn1_ref, jko_ref, *, first):
    dis = _dis_from(degp_ref[...])
    hp = jnp.concatenate([hp0_ref[...], hp1_ref[...]], axis=-1)
    o = dis[:, None] * (_cat(acc_ref) + hp) + b_ref[...]
    xn = _ln_relu(x_ref[...] + o, g_ref[...], be_ref[...])
    xn_ref[...] = xn
    hpn = jnp.dot(xn, wn_ref[...],
                  preferred_element_type=jnp.float32) * dis[:, None]
    hpn0_ref[...] = hpn[:, :DH]
    hpn1_ref[...] = hpn[:, DH:]
    jk = jnp.dot(xn, wjk_ref[...], preferred_element_type=jnp.float32)
    if not first:
        jk = jk + jk_ref[...]
    jko_ref[...] = jk


def _tc_final_body(degp_ref, x_ref, hp0_ref, hp1_ref, acc_ref,
                   b_ref, g_ref, be_ref, wjk_ref, bjk_ref, jk_ref, out_ref):
    dis = _dis_from(degp_ref[...])
    hp = jnp.concatenate([hp0_ref[...], hp1_ref[...]], axis=-1)
    o = dis[:, None] * (_cat(acc_ref) + hp) + b_ref[...]
    xn = _ln_relu(x_ref[...] + o, g_ref[...], be_ref[...])
    out_ref[...] = (jk_ref[...] + bjk_ref[...]
                    + jnp.dot(xn, wjk_ref[...],
                              preferred_element_type=jnp.float32))


_b_degp = pl.BlockSpec((BN, NW), lambda i: (i, 0))
_b_rows = pl.BlockSpec((BN, D), lambda i: (i, 0))
_b_hrow = pl.BlockSpec((BN, DH), lambda i: (i, 0))
_b_acc = pl.BlockSpec((NC, BN, DH), lambda i: (0, i, 0))
_b_w = pl.BlockSpec((D, D), lambda i: (0, 0))
_b_vec = pl.BlockSpec((1, D), lambda i: (0, 0))

_f32 = jnp.float32
_nd = jax.ShapeDtypeStruct((N, D), _f32)
_nh = jax.ShapeDtypeStruct((N, DH), _f32)

_tc_prep = pl.pallas_call(
    _tc_prep_body,
    grid=(GRID,),
    in_specs=[_b_degp, _b_rows, _b_w],
    out_specs=[_b_hrow, _b_hrow],
    out_shape=[_nh, _nh],
)


def _make_post(first):
    return pl.pallas_call(
        functools.partial(_tc_post_body, first=first),
        grid=(GRID,),
        in_specs=[_b_degp, _b_rows, _b_hrow, _b_hrow, _b_acc,
                  _b_vec, _b_vec, _b_vec, _b_w, _b_w, _b_rows],
        out_specs=[_b_rows, _b_hrow, _b_hrow, _b_rows],
        out_shape=[_nd, _nh, _nh, _nd],
    )


_tc_post0 = _make_post(True)
_tc_post1 = _make_post(False)

_tc_final = pl.pallas_call(
    _tc_final_body,
    grid=(GRID,),
    in_specs=[_b_degp, _b_rows, _b_hrow, _b_hrow, _b_acc,
              _b_vec, _b_vec, _b_vec, _b_w, _b_vec, _b_rows],
    out_specs=_b_rows,
    out_shape=_nd,
)


def kernel(node, edge_index, edge_attr, batch_ptr,
           W0, b0, g0, be0, W1, b1, g1, be1, W2, b2, g2, be2,
           Wjk, bjk):
    del batch_ptr
    row = edge_index[0].astype(jnp.int32)
    col = edge_index[1].astype(jnp.int32)
    w = edge_attr.astype(jnp.float32)

    degp = _deg_kernel(col, w).T

    pad = EPAD - E
    rowp = jnp.concatenate([row, jnp.zeros((pad,), jnp.int32)])
    colp = jnp.concatenate([col, jnp.zeros((pad,), jnp.int32)])
    wp = jnp.concatenate([w, jnp.zeros((pad,), jnp.float32)])
    rc = jnp.stack([rowp.reshape(NS * CPB, CH),
                    colp.reshape(NS * CPB, CH)], axis=1)
    wchunk = wp.reshape(NS * CPB, CH)

    b0r, g0r, be0r = b0.reshape(1, D), g0.reshape(1, D), be0.reshape(1, D)
    b1r, g1r, be1r = b1.reshape(1, D), g1.reshape(1, D), be1.reshape(1, D)
    b2r, g2r, be2r = b2.reshape(1, D), g2.reshape(1, D), be2.reshape(1, D)
    wjk0, wjk1, wjk2 = Wjk[:D], Wjk[D:2 * D], Wjk[2 * D:]
    bjkr = bjk.reshape(1, D)

    hp0a, hp0b = _tc_prep(degp, node, W0)
    acc0 = _agg_kernel(hp0a, hp0b, rc, wchunk)
    x1, hp1a, hp1b, jk = _tc_post0(degp, node, hp0a, hp0b, acc0,
                                   b0r, g0r, be0r, W1, wjk0,
                                   jnp.zeros((N, D), _f32))
    acc1 = _agg_kernel(hp1a, hp1b, rc, wchunk)
    x2, hp2a, hp2b, jk = _tc_post1(degp, x1, hp1a, hp1b, acc1,
                                   b1r, g1r, be1r, W2, wjk1, jk)
    acc2 = _agg_kernel(hp2a, hp2b, rc, wchunk)
    return _tc_final(degp, x2, hp2a, hp2b, acc2,
                     b2r, g2r, be2r, wjk2, bjkr, jk)
